# Initial kernel scaffold; baseline (speedup 1.0000x reference)
import jax
import jax.numpy as jnp
from jax.experimental import pallas as pl


def kernel(hidden_states, W, scale, per_expert_scale):
    raise NotImplementedError("write your pallas kernel here")


# speedup vs baseline: 1.7796x; 1.7796x over previous
import functools

import jax
import jax.numpy as jnp
from jax.experimental import pallas as pl
from jax.experimental.pallas import tpu as pltpu

HIDDEN = 2048
EXPERTS = 64
TOPK = 8
EPS = 1e-6
BLOCK = 512


def _router_block(h_ref, w_ref, sv_ref, pes_ref, wout_ref, iout_ref, cnt_ref):
    pid = pl.program_id(0)
    h = h_ref[...]
    var = jnp.mean(h * h, axis=1, keepdims=True)
    hn = h * jax.lax.rsqrt(var + EPS)
    hn = hn * sv_ref[...]
    logits = jnp.dot(hn, w_ref[...], preferred_element_type=jnp.float32)
    mx = jnp.max(logits, axis=1, keepdims=True)
    ex = jnp.exp(logits - mx)
    p = ex / jnp.sum(ex, axis=1, keepdims=True)

    iota = jax.lax.broadcasted_iota(jnp.int32, (BLOCK, EXPERTS), 1)
    pes = pes_ref[...]
    ws, idxs, pss = [], [], []
    onehot_acc = jnp.zeros((BLOCK, EXPERTS), jnp.float32)
    for _ in range(TOPK):
        mval = jnp.max(p, axis=1, keepdims=True)
        eq = p == mval
        idx = jnp.min(jnp.where(eq, iota, EXPERTS), axis=1, keepdims=True)
        sel = iota == idx
        ws.append(mval)
        idxs.append(idx)
        pss.append(jnp.sum(jnp.where(sel, pes, 0.0), axis=1, keepdims=True))
        onehot_acc = onehot_acc + sel.astype(jnp.float32)
        p = jnp.where(sel, -jnp.inf, p)

    w = jnp.concatenate(ws, axis=1)
    i = jnp.concatenate(idxs, axis=1)
    psel = jnp.concatenate(pss, axis=1)
    w = w / jnp.sum(w, axis=1, keepdims=True) * psel

    wout_ref[...] = w
    iout_ref[...] = i

    @pl.when(pid == 0)
    def _():
        cnt_ref[...] = jnp.zeros_like(cnt_ref)

    cnt_ref[...] += jnp.sum(onehot_acc, axis=0, keepdims=True)


@jax.jit
def kernel(hidden_states, W, scale, per_expert_scale):
    tokens = hidden_states.shape[0]
    grid = tokens // BLOCK
    sv = (scale * (HIDDEN ** -0.5)).reshape(1, HIDDEN)
    wt = W.T
    pes = per_expert_scale.reshape(1, EXPERTS)
    wout, iout, cnt = pl.pallas_call(
        _router_block,
        grid=(grid,),
        in_specs=[
            pl.BlockSpec((BLOCK, HIDDEN), lambda i: (i, 0)),
            pl.BlockSpec((HIDDEN, EXPERTS), lambda i: (0, 0)),
            pl.BlockSpec((1, HIDDEN), lambda i: (0, 0)),
            pl.BlockSpec((1, EXPERTS), lambda i: (0, 0)),
        ],
        out_specs=[
            pl.BlockSpec((BLOCK, TOPK), lambda i: (i, 0)),
            pl.BlockSpec((BLOCK, TOPK), lambda i: (i, 0)),
            pl.BlockSpec((1, EXPERTS), lambda i: (0, 0)),
        ],
        out_shape=[
            jax.ShapeDtypeStruct((tokens, TOPK), jnp.float32),
            jax.ShapeDtypeStruct((tokens, TOPK), jnp.int32),
            jax.ShapeDtypeStruct((1, EXPERTS), jnp.float32),
        ],
        compiler_params=pltpu.CompilerParams(
            dimension_semantics=("arbitrary",),
        ),
    )(hidden_states, wt, sv, pes)
    return wout, iout, cnt.reshape(EXPERTS)
